# baseline (speedup 1.0000x reference)
import functools

import jax
import jax.numpy as jnp
from jax import lax
from jax.experimental import pallas as pl
from jax.experimental.pallas import tpu as pltpu
from jax.experimental.pallas import tpu_sc as plsc

NUM_LEVELS = 16
BASE_RES = 16
MAX_RES = 2048
FEATS = 2
HASH_SIZE = 524288
D0, D1 = 64, 32
N_PTS = 1048576


def _level_params(level):
    resolution = int(BASE_RES * (MAX_RES / BASE_RES) ** (level / (NUM_LEVELS - 1)))
    hash_size = min(resolution ** 3, HASH_SIZE)
    return resolution, hash_size


_LEVELS = [_level_params(l) for l in range(NUM_LEVELS)]


_HASH_BN = 8192


def _hash_body(coords_ref, idx_ref):
    c = (coords_ref[...] + 1.0) / 2.0
    for lvl in range(NUM_LEVELS):
        res, hs = _LEVELS[lvl]
        cd = jnp.clip(jnp.floor(c * res).astype(jnp.int32), 0, res - 1)
        x, y, z = cd[0], cd[1], cd[2]
        if hs == res ** 3:
            idx = (x * res) * res + y * res + z
        else:
            m = hs - 1
            idx = (((x * res) & m) * res + y * res + z) & m
        e = idx * 2
        idx_ref[0, lvl, :] = e
        idx_ref[1, lvl, :] = e + 1


def _hash_indices(coords_t):
    grid = N_PTS // _HASH_BN
    return pl.pallas_call(
        _hash_body,
        grid=(grid,),
        in_specs=[pl.BlockSpec((3, _HASH_BN), lambda i: (0, i))],
        out_specs=pl.BlockSpec((2, NUM_LEVELS, _HASH_BN), lambda i: (0, 0, i)),
        out_shape=jax.ShapeDtypeStruct((2, NUM_LEVELS, N_PTS), jnp.int32),
    )(coords_t)


_NC = 2
_NS = 16
_NW = _NC * _NS
_BPW = N_PTS // _NW
_GCH = 2048
_NBUF = 4


def _gather_body(idx_hbm, *rest):
    tables = rest[:NUM_LEVELS]
    out_hbm = rest[NUM_LEVELS]
    scratch = rest[NUM_LEVELS + 1:]
    tbl_s = scratch[0]
    idxe_v = scratch[1:1 + _NBUF]
    idxo_v = scratch[1 + _NBUF:1 + 2 * _NBUF]
    e_v = scratch[1 + 2 * _NBUF:1 + 3 * _NBUF]
    o_v = scratch[1 + 3 * _NBUF:1 + 4 * _NBUF]
    isem = scratch[1 + 4 * _NBUF:1 + 5 * _NBUF]
    gsem = scratch[1 + 5 * _NBUF:1 + 6 * _NBUF]
    osem = scratch[1 + 6 * _NBUF:1 + 7 * _NBUF]
    sid = lax.axis_index("s")
    wid = sid * _NC + lax.axis_index("c")
    base = wid * _BPW
    nch = _BPW // _GCH

    def _fire_idx(lvl, j, b):
        off = base + j * _GCH
        pltpu.async_copy(idx_hbm.at[pl.ds(lvl * N_PTS + off, _GCH)],
                         idxe_v[b], isem[b])
        pltpu.async_copy(idx_hbm.at[pl.ds((NUM_LEVELS + lvl) * N_PTS + off, _GCH)],
                         idxo_v[b], isem[b])

    def _wait_idx(lvl, j, b):
        off = base + j * _GCH
        pltpu.make_async_copy(idx_hbm.at[pl.ds(lvl * N_PTS + off, _GCH)],
                              idxe_v[b], isem[b]).wait()
        pltpu.make_async_copy(
            idx_hbm.at[pl.ds((NUM_LEVELS + lvl) * N_PTS + off, _GCH)],
            idxo_v[b], isem[b]).wait()

    for b in range(_NBUF):
        _fire_idx(0, b, b)

    for lvl in range(NUM_LEVELS):
        _, hs = _LEVELS[lvl]
        nrows = -(-2 * hs // 128) * 128
        plsc.subcore_barrier()

        @pl.when(sid == 0)
        def _stage(lvl=lvl, nrows=nrows):
            pltpu.sync_copy(tables[lvl], tbl_s.at[pl.ds(0, nrows)])

        plsc.subcore_barrier()

        @pl.loop(0, nch, step=_NBUF)
        def _group(j0, lvl=lvl):
            for b in range(_NBUF):
                j = j0 + b
                off = base + j * _GCH
                _wait_idx(lvl, j, b)
                pltpu.async_copy(tbl_s.at[idxe_v[b]], e_v[b], gsem[b])
                pltpu.async_copy(tbl_s.at[idxo_v[b]], o_v[b], gsem[b])
            for b in range(_NBUF):
                j = j0 + b
                off = base + j * _GCH
                pltpu.make_async_copy(tbl_s.at[idxe_v[b]], e_v[b], gsem[b]).wait()
                pltpu.make_async_copy(tbl_s.at[idxo_v[b]], o_v[b], gsem[b]).wait()
                nj = j + _NBUF
                nlvl = lvl + 1 if lvl + 1 < NUM_LEVELS else lvl
                if lvl + 1 < NUM_LEVELS or True:
                    @pl.when(nj < nch)
                    def _pf_same(j=j, b=b, lvl=lvl):
                        _fire_idx(lvl, j + _NBUF, b)
                    if lvl + 1 < NUM_LEVELS:
                        @pl.when(nj >= nch)
                        def _pf_next(j=j, b=b, nlvl=nlvl):
                            _fire_idx(nlvl, j + _NBUF - nch, b)
                pltpu.async_copy(
                    e_v[b], out_hbm.at[pl.ds(lvl * N_PTS + off, _GCH)], osem[b])
                pltpu.async_copy(
                    o_v[b],
                    out_hbm.at[pl.ds((NUM_LEVELS + lvl) * N_PTS + off, _GCH)],
                    osem[b])
            for b in range(_NBUF):
                off = base + (j0 + b) * _GCH
                pltpu.make_async_copy(
                    e_v[b], out_hbm.at[pl.ds(lvl * N_PTS + off, _GCH)],
                    osem[b]).wait()
                pltpu.make_async_copy(
                    o_v[b],
                    out_hbm.at[pl.ds((NUM_LEVELS + lvl) * N_PTS + off, _GCH)],
                    osem[b]).wait()


def _sc_gather(idx, tables_flat):
    mesh = plsc.VectorSubcoreMesh(core_axis_name="c", subcore_axis_name="s")
    k = functools.partial(
        pl.kernel,
        mesh=mesh,
        out_type=jax.ShapeDtypeStruct((FEATS * NUM_LEVELS * N_PTS,), jnp.float32),
        scratch_types=(
            [pltpu.VMEM_SHARED((2 * HASH_SIZE,), jnp.float32)]
            + [pltpu.VMEM((_GCH,), jnp.int32) for _ in range(2 * _NBUF)]
            + [pltpu.VMEM((_GCH,), jnp.float32) for _ in range(2 * _NBUF)]
            + [pltpu.SemaphoreType.DMA for _ in range(3 * _NBUF)]
        ),
    )(_gather_body)
    return k(idx, *tables_flat)


_MLP_BN = 2048


def _mlp_body(f_ref, w0_ref, b0_ref, w1_ref, b1_ref, out_ref):
    xt = f_ref[...].reshape(NUM_LEVELS * FEATS, _MLP_BN)
    h = jnp.maximum(
        lax.dot_general(xt, w0_ref[...], (((0,), (0,)), ((), ())),
                        preferred_element_type=jnp.float32) + b0_ref[...],
        0.0,
    )
    out_ref[...] = (
        jnp.dot(h, w1_ref[...], preferred_element_type=jnp.float32) + b1_ref[...]
    )


def _mlp(feats, W0, b0, W1, b1):
    W0p = W0.reshape(NUM_LEVELS, FEATS, D0).transpose(1, 0, 2).reshape(
        NUM_LEVELS * FEATS, D0)
    grid = N_PTS // _MLP_BN
    return pl.pallas_call(
        _mlp_body,
        grid=(grid,),
        in_specs=[
            pl.BlockSpec((FEATS, NUM_LEVELS, _MLP_BN), lambda i: (0, 0, i)),
            pl.BlockSpec((NUM_LEVELS * FEATS, D0), lambda i: (0, 0)),
            pl.BlockSpec((D0,), lambda i: (0,)),
            pl.BlockSpec((D0, D1), lambda i: (0, 0)),
            pl.BlockSpec((D1,), lambda i: (0,)),
        ],
        out_specs=pl.BlockSpec((_MLP_BN, D1), lambda i: (i, 0)),
        out_shape=jax.ShapeDtypeStruct((N_PTS, D1), jnp.float32),
    )(feats, W0p, b0, W1, b1)


def kernel(coords, W0, b0, W1, b1, table_0, table_1, table_2, table_3,
           table_4, table_5, table_6, table_7, table_8, table_9, table_10,
           table_11, table_12, table_13, table_14, table_15):
    tables = [table_0, table_1, table_2, table_3, table_4, table_5, table_6,
              table_7, table_8, table_9, table_10, table_11, table_12,
              table_13, table_14, table_15]
    idx = _hash_indices(coords.T)
    tf = [t.reshape(-1) for t in tables]
    tf = [jnp.pad(t, (0, -len(t) % 128)) if len(t) % 128 else t for t in tf]
    flat = _sc_gather(idx.reshape(-1), tf)
    feats = flat.reshape(FEATS, NUM_LEVELS, N_PTS)
    return _mlp(feats, W0, b0, W1, b1)


# speedup vs baseline: 1.5402x; 1.5402x over previous
import functools

import jax
import jax.numpy as jnp
from jax import lax
from jax.experimental import pallas as pl
from jax.experimental.pallas import tpu as pltpu
from jax.experimental.pallas import tpu_sc as plsc

NUM_LEVELS = 16
BASE_RES = 16
MAX_RES = 2048
FEATS = 2
HASH_SIZE = 524288
D0, D1 = 64, 32
N_PTS = 1048576


def _level_params(level):
    resolution = int(BASE_RES * (MAX_RES / BASE_RES) ** (level / (NUM_LEVELS - 1)))
    hash_size = min(resolution ** 3, HASH_SIZE)
    return resolution, hash_size


_LEVELS = [_level_params(l) for l in range(NUM_LEVELS)]


_HASH_BN = 8192


def _hash_body(coords_ref, idx_ref):
    c = (coords_ref[...] + 1.0) / 2.0
    for lvl in range(NUM_LEVELS):
        res, hs = _LEVELS[lvl]
        cd = jnp.clip(jnp.floor(c * res).astype(jnp.int32), 0, res - 1)
        x, y, z = cd[0], cd[1], cd[2]
        if hs == res ** 3:
            idx = (x * res) * res + y * res + z
        else:
            m = hs - 1
            idx = (((x * res) & m) * res + y * res + z) & m
        e = idx * 2
        idx_ref[0, lvl, :] = e
        idx_ref[1, lvl, :] = e + 1


def _hash_indices(coords_t):
    grid = N_PTS // _HASH_BN
    return pl.pallas_call(
        _hash_body,
        grid=(grid,),
        in_specs=[pl.BlockSpec((3, _HASH_BN), lambda i: (0, i))],
        out_specs=pl.BlockSpec((2, NUM_LEVELS, _HASH_BN), lambda i: (0, 0, i)),
        out_shape=jax.ShapeDtypeStruct((2, NUM_LEVELS, N_PTS), jnp.int32),
    )(coords_t)


_NC = 2
_NS = 16
_NW = _NC * _NS
_BPW = N_PTS // _NW
_GCH = 2048
_NBUF = 4


def _gather_body(idx_hbm, *rest):
    tables = rest[:NUM_LEVELS]
    out_hbm = rest[NUM_LEVELS]
    scratch = rest[NUM_LEVELS + 1:]
    tbl_s = scratch[0]
    idxe_v = scratch[1:1 + _NBUF]
    idxo_v = scratch[1 + _NBUF:1 + 2 * _NBUF]
    e_v = scratch[1 + 2 * _NBUF:1 + 3 * _NBUF]
    o_v = scratch[1 + 3 * _NBUF:1 + 4 * _NBUF]
    isem = scratch[1 + 4 * _NBUF:1 + 5 * _NBUF]
    gsem = scratch[1 + 5 * _NBUF:1 + 6 * _NBUF]
    osem = scratch[1 + 6 * _NBUF:1 + 7 * _NBUF]
    sid = lax.axis_index("s")
    wid = sid * _NC + lax.axis_index("c")
    base = wid * _BPW
    nch = _BPW // _GCH

    def _fire_idx(lvl, j, b):
        off = base + j * _GCH
        pltpu.async_copy(idx_hbm.at[pl.ds(lvl * N_PTS + off, _GCH)],
                         idxe_v[b], isem[b])
        pltpu.async_copy(idx_hbm.at[pl.ds((NUM_LEVELS + lvl) * N_PTS + off, _GCH)],
                         idxo_v[b], isem[b])

    def _wait_idx(lvl, j, b):
        off = base + j * _GCH
        pltpu.make_async_copy(idx_hbm.at[pl.ds(lvl * N_PTS + off, _GCH)],
                              idxe_v[b], isem[b]).wait()
        pltpu.make_async_copy(
            idx_hbm.at[pl.ds((NUM_LEVELS + lvl) * N_PTS + off, _GCH)],
            idxo_v[b], isem[b]).wait()

    for b in range(_NBUF):
        _fire_idx(0, b, b)

    for lvl in range(NUM_LEVELS):
        _, hs = _LEVELS[lvl]
        nrows = -(-2 * hs // 128) * 128
        plsc.subcore_barrier()

        @pl.when(sid == 0)
        def _stage(lvl=lvl, nrows=nrows):
            pltpu.sync_copy(tables[lvl], tbl_s.at[pl.ds(0, nrows)])

        plsc.subcore_barrier()

        @pl.loop(0, nch, step=_NBUF)
        def _group(j0, lvl=lvl):
            for b in range(_NBUF):
                j = j0 + b
                off = base + j * _GCH
                _wait_idx(lvl, j, b)
                pltpu.async_copy(tbl_s.at[idxe_v[b]], e_v[b], gsem[b])
                pltpu.async_copy(tbl_s.at[idxo_v[b]], o_v[b], gsem[b])
            for b in range(_NBUF):
                j = j0 + b
                off = base + j * _GCH
                pltpu.make_async_copy(tbl_s.at[idxe_v[b]], e_v[b], gsem[b]).wait()
                pltpu.make_async_copy(tbl_s.at[idxo_v[b]], o_v[b], gsem[b]).wait()
                nj = j + _NBUF
                nlvl = lvl + 1 if lvl + 1 < NUM_LEVELS else lvl
                if lvl + 1 < NUM_LEVELS or True:
                    @pl.when(nj < nch)
                    def _pf_same(j=j, b=b, lvl=lvl):
                        _fire_idx(lvl, j + _NBUF, b)
                    if lvl + 1 < NUM_LEVELS:
                        @pl.when(nj >= nch)
                        def _pf_next(j=j, b=b, nlvl=nlvl):
                            _fire_idx(nlvl, j + _NBUF - nch, b)
                pltpu.async_copy(
                    e_v[b], out_hbm.at[pl.ds(lvl * N_PTS + off, _GCH)], osem[b])
                pltpu.async_copy(
                    o_v[b],
                    out_hbm.at[pl.ds((NUM_LEVELS + lvl) * N_PTS + off, _GCH)],
                    osem[b])
            for b in range(_NBUF):
                off = base + (j0 + b) * _GCH
                pltpu.make_async_copy(
                    e_v[b], out_hbm.at[pl.ds(lvl * N_PTS + off, _GCH)],
                    osem[b]).wait()
                pltpu.make_async_copy(
                    o_v[b],
                    out_hbm.at[pl.ds((NUM_LEVELS + lvl) * N_PTS + off, _GCH)],
                    osem[b]).wait()


def _sc_gather(idx, tables_flat):
    mesh = plsc.VectorSubcoreMesh(core_axis_name="c", subcore_axis_name="s")
    k = functools.partial(
        pl.kernel,
        mesh=mesh,
        out_type=jax.ShapeDtypeStruct((FEATS * NUM_LEVELS * N_PTS,), jnp.float32),
        scratch_types=(
            [pltpu.VMEM_SHARED((2 * HASH_SIZE,), jnp.float32)]
            + [pltpu.VMEM((_GCH,), jnp.int32) for _ in range(2 * _NBUF)]
            + [pltpu.VMEM((_GCH,), jnp.float32) for _ in range(2 * _NBUF)]
            + [pltpu.SemaphoreType.DMA for _ in range(3 * _NBUF)]
        ),
    )(_gather_body)
    return k(idx, *tables_flat)


_MLP_BN = 2048


def _mlp_body(*refs):
    f = refs[:NUM_LEVELS * FEATS]
    w0_ref, b0_ref, w1_ref, b1_ref, out_ref = refs[NUM_LEVELS * FEATS:]
    xt = jnp.stack([r[...] for r in f])
    h = jnp.maximum(
        lax.dot_general(xt, w0_ref[...], (((0,), (0,)), ((), ())),
                        preferred_element_type=jnp.float32) + b0_ref[...],
        0.0,
    )
    out_ref[...] = (
        jnp.dot(h, w1_ref[...], preferred_element_type=jnp.float32) + b1_ref[...]
    )


def _mlp(flat, W0, b0, W1, b1):
    W0p = W0.reshape(NUM_LEVELS, FEATS, D0).transpose(1, 0, 2).reshape(
        NUM_LEVELS * FEATS, D0)
    grid = N_PTS // _MLP_BN
    return pl.pallas_call(
        _mlp_body,
        grid=(grid,),
        in_specs=(
            [pl.BlockSpec((_MLP_BN,), (lambda i, k=k: (k * (N_PTS // _MLP_BN) + i,)))
             for k in range(NUM_LEVELS * FEATS)]
            + [
                pl.BlockSpec((NUM_LEVELS * FEATS, D0), lambda i: (0, 0)),
                pl.BlockSpec((D0,), lambda i: (0,)),
                pl.BlockSpec((D0, D1), lambda i: (0, 0)),
                pl.BlockSpec((D1,), lambda i: (0,)),
            ]
        ),
        out_specs=pl.BlockSpec((_MLP_BN, D1), lambda i: (i, 0)),
        out_shape=jax.ShapeDtypeStruct((N_PTS, D1), jnp.float32),
    )(*([flat] * (NUM_LEVELS * FEATS)), W0p, b0, W1, b1)


def kernel(coords, W0, b0, W1, b1, table_0, table_1, table_2, table_3,
           table_4, table_5, table_6, table_7, table_8, table_9, table_10,
           table_11, table_12, table_13, table_14, table_15):
    tables = [table_0, table_1, table_2, table_3, table_4, table_5, table_6,
              table_7, table_8, table_9, table_10, table_11, table_12,
              table_13, table_14, table_15]
    idx = _hash_indices(coords.T)
    tf = [t.reshape(-1) for t in tables]
    tf = [jnp.pad(t, (0, -len(t) % 128)) if len(t) % 128 else t for t in tf]
    flat = _sc_gather(idx.reshape(-1), tf)
    return _mlp(flat, W0, b0, W1, b1)


# speedup vs baseline: 1.6998x; 1.1037x over previous
import functools

import jax
import jax.numpy as jnp
from jax import lax
from jax.experimental import pallas as pl
from jax.experimental.pallas import tpu as pltpu
from jax.experimental.pallas import tpu_sc as plsc

NUM_LEVELS = 16
BASE_RES = 16
MAX_RES = 2048
FEATS = 2
HASH_SIZE = 524288
D0, D1 = 64, 32
N_PTS = 1048576


def _level_params(level):
    resolution = int(BASE_RES * (MAX_RES / BASE_RES) ** (level / (NUM_LEVELS - 1)))
    hash_size = min(resolution ** 3, HASH_SIZE)
    return resolution, hash_size


_LEVELS = [_level_params(l) for l in range(NUM_LEVELS)]


_HASH_BN = 8192


def _hash_body(coords_ref, *idx_ref):
    c = (coords_ref[...] + 1.0) / 2.0
    for lvl in range(NUM_LEVELS):
        res, hs = _LEVELS[lvl]
        cd = jnp.clip(jnp.floor(c * res).astype(jnp.int32), 0, res - 1)
        x, y, z = cd[0], cd[1], cd[2]
        if hs == res ** 3:
            idx = (x * res) * res + y * res + z
        else:
            m = hs - 1
            idx = (((x * res) & m) * res + y * res + z) & m
        e = idx * 2
        idx_ref[lvl][...] = e
        idx_ref[NUM_LEVELS + lvl][...] = e + 1


def _hash_indices(coords_t):
    grid = N_PTS // _HASH_BN
    return pl.pallas_call(
        _hash_body,
        grid=(grid,),
        in_specs=[pl.BlockSpec((3, _HASH_BN), lambda i: (0, i))],
        out_specs=[pl.BlockSpec((_HASH_BN,), lambda i: (i,))
                   for _ in range(2 * NUM_LEVELS)],
        out_shape=[jax.ShapeDtypeStruct((N_PTS,), jnp.int32)
                   for _ in range(2 * NUM_LEVELS)],
    )(coords_t)


_NC = 2
_NS = 16
_NW = _NC * _NS
_BPW = N_PTS // _NW
_GCH = 2048
_NBUF = 4


def _gather_body(*rest):
    idx_p = rest[:2 * NUM_LEVELS]
    rest = rest[2 * NUM_LEVELS:]
    tables = rest[:NUM_LEVELS]
    out_hbm = rest[NUM_LEVELS]
    scratch = rest[NUM_LEVELS + 1:]
    tbl_s = scratch[0]
    idxe_v = scratch[1:1 + _NBUF]
    idxo_v = scratch[1 + _NBUF:1 + 2 * _NBUF]
    e_v = scratch[1 + 2 * _NBUF:1 + 3 * _NBUF]
    o_v = scratch[1 + 3 * _NBUF:1 + 4 * _NBUF]
    isem = scratch[1 + 4 * _NBUF:1 + 5 * _NBUF]
    gsem = scratch[1 + 5 * _NBUF:1 + 6 * _NBUF]
    osem = scratch[1 + 6 * _NBUF:1 + 7 * _NBUF]
    sid = lax.axis_index("s")
    wid = sid * _NC + lax.axis_index("c")
    base = wid * _BPW
    nch = _BPW // _GCH

    def _fire_idx(lvl, j, b):
        off = base + j * _GCH
        pltpu.async_copy(idx_p[lvl].at[pl.ds(off, _GCH)], idxe_v[b], isem[b])
        pltpu.async_copy(idx_p[NUM_LEVELS + lvl].at[pl.ds(off, _GCH)],
                         idxo_v[b], isem[b])

    def _wait_idx(lvl, j, b):
        off = base + j * _GCH
        pltpu.make_async_copy(idx_p[lvl].at[pl.ds(off, _GCH)], idxe_v[b],
                              isem[b]).wait()
        pltpu.make_async_copy(idx_p[NUM_LEVELS + lvl].at[pl.ds(off, _GCH)],
                              idxo_v[b], isem[b]).wait()

    for b in range(_NBUF):
        _fire_idx(0, b, b)

    for lvl in range(NUM_LEVELS):
        _, hs = _LEVELS[lvl]
        nrows = -(-2 * hs // 128) * 128
        plsc.subcore_barrier()

        @pl.when(sid == 0)
        def _stage(lvl=lvl, nrows=nrows):
            pltpu.sync_copy(tables[lvl], tbl_s.at[pl.ds(0, nrows)])

        plsc.subcore_barrier()

        @pl.loop(0, nch, step=_NBUF)
        def _group(j0, lvl=lvl):
            for b in range(_NBUF):
                j = j0 + b
                off = base + j * _GCH
                _wait_idx(lvl, j, b)
                pltpu.async_copy(tbl_s.at[idxe_v[b]], e_v[b], gsem[b])
                pltpu.async_copy(tbl_s.at[idxo_v[b]], o_v[b], gsem[b])
            for b in range(_NBUF):
                j = j0 + b
                off = base + j * _GCH
                pltpu.make_async_copy(tbl_s.at[idxe_v[b]], e_v[b], gsem[b]).wait()
                pltpu.make_async_copy(tbl_s.at[idxo_v[b]], o_v[b], gsem[b]).wait()
                nj = j + _NBUF
                nlvl = lvl + 1 if lvl + 1 < NUM_LEVELS else lvl
                if lvl + 1 < NUM_LEVELS or True:
                    @pl.when(nj < nch)
                    def _pf_same(j=j, b=b, lvl=lvl):
                        _fire_idx(lvl, j + _NBUF, b)
                    if lvl + 1 < NUM_LEVELS:
                        @pl.when(nj >= nch)
                        def _pf_next(j=j, b=b, nlvl=nlvl):
                            _fire_idx(nlvl, j + _NBUF - nch, b)
                pltpu.async_copy(
                    e_v[b], out_hbm.at[pl.ds(lvl * N_PTS + off, _GCH)], osem[b])
                pltpu.async_copy(
                    o_v[b],
                    out_hbm.at[pl.ds((NUM_LEVELS + lvl) * N_PTS + off, _GCH)],
                    osem[b])
            for b in range(_NBUF):
                off = base + (j0 + b) * _GCH
                pltpu.make_async_copy(
                    e_v[b], out_hbm.at[pl.ds(lvl * N_PTS + off, _GCH)],
                    osem[b]).wait()
                pltpu.make_async_copy(
                    o_v[b],
                    out_hbm.at[pl.ds((NUM_LEVELS + lvl) * N_PTS + off, _GCH)],
                    osem[b]).wait()


def _sc_gather(idx, tables_flat):
    mesh = plsc.VectorSubcoreMesh(core_axis_name="c", subcore_axis_name="s")
    k = functools.partial(
        pl.kernel,
        mesh=mesh,
        out_type=jax.ShapeDtypeStruct((FEATS * NUM_LEVELS * N_PTS,), jnp.float32),
        scratch_types=(
            [pltpu.VMEM_SHARED((2 * HASH_SIZE,), jnp.float32)]
            + [pltpu.VMEM((_GCH,), jnp.int32) for _ in range(2 * _NBUF)]
            + [pltpu.VMEM((_GCH,), jnp.float32) for _ in range(2 * _NBUF)]
            + [pltpu.SemaphoreType.DMA for _ in range(3 * _NBUF)]
        ),
    )(_gather_body)
    return k(*idx, *tables_flat)


_MLP_BN = 2048


def _mlp_body(*refs):
    f = refs[:NUM_LEVELS * FEATS]
    w0_ref, b0_ref, w1_ref, b1_ref, out_ref = refs[NUM_LEVELS * FEATS:]
    xt = jnp.stack([r[...] for r in f])
    ht = jnp.maximum(
        lax.dot_general(w0_ref[...], xt, (((0,), (0,)), ((), ())),
                        preferred_element_type=jnp.float32)
        + b0_ref[...][:, None],
        0.0,
    )
    out_ref[...] = (
        lax.dot_general(w1_ref[...], ht, (((0,), (0,)), ((), ())),
                        preferred_element_type=jnp.float32)
        + b1_ref[...][:, None]
    )


def _mlp(flat, W0, b0, W1, b1):
    W0p = W0.reshape(NUM_LEVELS, FEATS, D0).transpose(1, 0, 2).reshape(
        NUM_LEVELS * FEATS, D0)
    grid = N_PTS // _MLP_BN
    return pl.pallas_call(
        _mlp_body,
        grid=(grid,),
        in_specs=(
            [pl.BlockSpec((_MLP_BN,), (lambda i, k=k: (k * (N_PTS // _MLP_BN) + i,)))
             for k in range(NUM_LEVELS * FEATS)]
            + [
                pl.BlockSpec((NUM_LEVELS * FEATS, D0), lambda i: (0, 0)),
                pl.BlockSpec((D0,), lambda i: (0,)),
                pl.BlockSpec((D0, D1), lambda i: (0, 0)),
                pl.BlockSpec((D1,), lambda i: (0,)),
            ]
        ),
        out_specs=pl.BlockSpec((D1, _MLP_BN), lambda i: (0, i)),
        out_shape=jax.ShapeDtypeStruct((D1, N_PTS), jnp.float32),
    )(*([flat] * (NUM_LEVELS * FEATS)), W0p, b0, W1, b1)


def kernel(coords, W0, b0, W1, b1, table_0, table_1, table_2, table_3,
           table_4, table_5, table_6, table_7, table_8, table_9, table_10,
           table_11, table_12, table_13, table_14, table_15):
    tables = [table_0, table_1, table_2, table_3, table_4, table_5, table_6,
              table_7, table_8, table_9, table_10, table_11, table_12,
              table_13, table_14, table_15]
    idx = _hash_indices(coords.T)
    tf = [t.reshape(-1) for t in tables]
    tf = [jnp.pad(t, (0, -len(t) % 128)) if len(t) % 128 else t for t in tf]
    flat = _sc_gather(idx, tf)
    return _mlp(flat, W0, b0, W1, b1).T
